# baseline (speedup 1.0000x reference)
import functools
import math

import jax
import jax.numpy as jnp
from jax import lax
from jax.experimental import pallas as pl
from jax.experimental.pallas import tpu as pltpu
from jax.experimental.pallas import tpu_sc as plsc

NI = 16384
NJ = 200
D = 32
N = NI * NJ
SCALE = math.sqrt(32.0)

_info = plsc.get_sparse_core_info()
NC = _info.num_cores
NS = _info.num_subcores
NW = NC * NS
L = _info.num_lanes

IB = 1024
HB = IB // 2
GATHER = 128
K = IB // GATHER
KH = K // 2
NBLK = NJ * (NI // IB)
PER_W = NBLK // NW
NPAIR = PER_W // 2
NIB = NI // IB

TROW = (KH + 1) * 8
TPITCH = 129

_mesh = plsc.VectorSubcoreMesh(core_axis_name="c", subcore_axis_name="s")


@functools.partial(
    pl.kernel,
    mesh=_mesh,
    out_type=jax.ShapeDtypeStruct((NJ, D // 8, (NI // 128) * 8, 128), jnp.float32),
    scratch_types=[
        pltpu.VMEM((K, GATHER), jnp.int32),
        pltpu.VMEM((K, GATHER), jnp.int32),
        pltpu.VMEM((IB, D), jnp.float32),
        pltpu.VMEM((IB, D), jnp.float32),
        pltpu.VMEM((4 * TROW, TPITCH), jnp.float32),
        pltpu.VMEM((4 * TROW, TPITCH), jnp.float32),
        pltpu.VMEM((128, L), jnp.int32),
        pltpu.SemaphoreType.DMA,
        pltpu.SemaphoreType.DMA,
        pltpu.SemaphoreType.DMA,
        pltpu.SemaphoreType.DMA,
        pltpu.SemaphoreType.DMA,
        pltpu.SemaphoreType.DMA,
    ],
    compiler_params=pltpu.CompilerParams(
        use_tc_tiling_on_sc=False, needs_layout_passes=False
    ),
)
def _embed(
    xt_hbm, table_hbm, out_hbm,
    idx_a, idx_b, rows_a, rows_b, tb0, tb1, ilv_tab,
    sem_a, sem_b, sem_o0, sem_o1, sem_ia, sem_ib,
):
    wid = lax.axis_index("s") * NC + lax.axis_index("c")
    base = wid * PER_W
    iota = lax.iota(jnp.int32, L)
    zero = iota * 0
    rowp0 = (iota // 8) * TROW + iota % 8
    rowp1 = rowp0 + 2 * TROW

    def _mk_ilv(il, carry):
        ilv_tab[il, pl.ds(0, L)] = zero + il
        return carry

    lax.fori_loop(0, 128, _mk_ilv, 0)

    def idx_copy(b, idx_v, sem):
        j = b // NIB
        ib = b % NIB
        return pltpu.make_async_copy(
            xt_hbm.at[j, pl.ds(ib * K, K)], idx_v, sem
        )

    def gather_copies(idx_v, rows_v, sem):
        for g in range(K):
            yield pltpu.make_async_copy(
                table_hbm.at[idx_v.at[g]],
                rows_v.at[pl.ds(g * GATHER, GATHER)],
                sem,
            )

    def fire_gathers(idx_v, rows_v, sem):
        for c in gather_copies(idx_v, rows_v, sem):
            c.start()

    def wait_gathers(idx_v, rows_v, sem):
        for c in gather_copies(idx_v, rows_v, sem):
            c.wait()

    def xpose_half(rows_v, tb, h):
        for sub in range(KH):
            t0 = h * HB + sub * 128
            rv0 = rowp0 + sub * 8
            rv1 = rowp1 + sub * 8

            def body(il, carry):
                ilv = ilv_tab[il, pl.ds(0, L)]
                t = t0 + il
                val0 = rows_v[t, pl.ds(0, L)] * SCALE
                val1 = rows_v[t, pl.ds(L, L)] * SCALE
                plsc.store_scatter(tb, [rv0, ilv], val0)
                plsc.store_scatter(tb, [rv1, ilv], val1)
                return carry

            lax.fori_loop(0, 128, body, 0, unroll=4)

    def out_copies(b, tb, h, sem):
        j = b // NIB
        ib = b % NIB
        for c4 in range(D // 8):
            yield pltpu.make_async_copy(
                tb.at[pl.ds(c4 * TROW, KH * 8), pl.ds(0, 128)],
                out_hbm.at[j, c4, pl.ds((ib * K + h * KH) * 8, KH * 8)],
                sem,
            )

    def fire_out(b, tb, h, sem):
        for c in out_copies(b, tb, h, sem):
            c.start()

    def wait_out(b, tb, h, sem):
        for c in out_copies(b, tb, h, sem):
            c.wait()

    def process(b, rows_v, first):
        @pl.when(jnp.logical_not(first))
        def _():
            wait_out(b, tb0, 0, sem_o0)

        xpose_half(rows_v, tb0, 0)
        fire_out(b, tb0, 0, sem_o0)

        @pl.when(jnp.logical_not(first))
        def _():
            wait_out(b, tb1, 1, sem_o1)

        xpose_half(rows_v, tb1, 1)
        fire_out(b, tb1, 1, sem_o1)

    idx_copy(base, idx_a, sem_ia).start()
    idx_copy(base, idx_a, sem_ia).wait()
    fire_gathers(idx_a, rows_a, sem_a)
    idx_copy(base + 1, idx_b, sem_ib).start()

    def pair_body(k, carry):
        b0 = base + 2 * k
        b1 = b0 + 1
        not_last = k + 1 < NPAIR

        idx_copy(b1, idx_b, sem_ib).wait()
        fire_gathers(idx_b, rows_b, sem_b)

        wait_gathers(idx_a, rows_a, sem_a)

        @pl.when(not_last)
        def _():
            idx_copy(b0 + 2, idx_a, sem_ia).start()

        process(b0, rows_a, k == 0)

        @pl.when(not_last)
        def _():
            idx_copy(b0 + 2, idx_a, sem_ia).wait()
            fire_gathers(idx_a, rows_a, sem_a)

        wait_gathers(idx_b, rows_b, sem_b)

        @pl.when(not_last)
        def _():
            idx_copy(b1 + 2, idx_b, sem_ib).start()

        process(b1, rows_b, False)
        return carry

    lax.fori_loop(0, NPAIR, pair_body, 0)
    wait_out(0, tb0, 0, sem_o0)
    wait_out(0, tb1, 1, sem_o1)


def kernel(x, table):
    xt = x.T.reshape(NJ, NI // 128, 128)
    o4 = _embed(xt, table)
    return (
        o4.reshape(NJ, D // 8, NI // 128, 8, 128)
        .transpose(2, 4, 0, 1, 3)
        .reshape(NI, NJ, D)
    )


# speedup vs baseline: 1.1212x; 1.1212x over previous
import functools
import math

import jax
import jax.numpy as jnp
from jax import lax
from jax.experimental import pallas as pl
from jax.experimental.pallas import tpu as pltpu
from jax.experimental.pallas import tpu_sc as plsc

NI = 16384
NJ = 200
D = 32
N = NI * NJ
SCALE = math.sqrt(32.0)

_info = plsc.get_sparse_core_info()
NC = _info.num_cores
NS = _info.num_subcores
NW = NC * NS
L = _info.num_lanes

IB = 1024
HB = IB // 2
GATHER = 128
K = IB // GATHER
KH = K // 2
NBLK = NJ * (NI // IB)
PER_W = NBLK // NW
NPAIR = PER_W // 2
NIB = NI // IB

TROW = (KH + 1) * 8
TPITCH = 129

_mesh = plsc.VectorSubcoreMesh(core_axis_name="c", subcore_axis_name="s")


@functools.partial(
    pl.kernel,
    mesh=_mesh,
    out_type=jax.ShapeDtypeStruct((NJ, D // 8, (NI // 128) * 8, 128), jnp.float32),
    scratch_types=[
        pltpu.VMEM((K, GATHER), jnp.int32),
        pltpu.VMEM((K, GATHER), jnp.int32),
        pltpu.VMEM((IB, D), jnp.float32),
        pltpu.VMEM((IB, D), jnp.float32),
        pltpu.VMEM((4 * TROW, TPITCH), jnp.float32),
        pltpu.VMEM((4 * TROW, TPITCH), jnp.float32),
        pltpu.VMEM((128, L), jnp.int32),
        pltpu.SemaphoreType.DMA,
        pltpu.SemaphoreType.DMA,
        pltpu.SemaphoreType.DMA,
        pltpu.SemaphoreType.DMA,
        pltpu.SemaphoreType.DMA,
        pltpu.SemaphoreType.DMA,
    ],
    compiler_params=pltpu.CompilerParams(
        use_tc_tiling_on_sc=False, needs_layout_passes=False
    ),
)
def _embed(
    xt_hbm, table_hbm, out_hbm,
    idx_a, idx_b, rows_a, rows_b, tb0, tb1, ilv_tab,
    sem_a, sem_b, sem_o0, sem_o1, sem_ia, sem_ib,
):
    wid = lax.axis_index("s") * NC + lax.axis_index("c")
    base = wid * PER_W
    iota = lax.iota(jnp.int32, L)
    zero = iota * 0
    rowp0 = (iota // 8) * TROW + iota % 8
    rowp1 = rowp0 + 2 * TROW

    def _mk_ilv(il, carry):
        ilv_tab[il, pl.ds(0, L)] = zero + il
        return carry

    lax.fori_loop(0, 128, _mk_ilv, 0)

    def idx_copy(b, idx_v, sem):
        j = b // NIB
        ib = b % NIB
        return pltpu.make_async_copy(
            xt_hbm.at[j, pl.ds(ib * K, K)], idx_v, sem
        )

    def gather_copies(idx_v, rows_v, sem):
        for g in range(K):
            yield pltpu.make_async_copy(
                table_hbm.at[idx_v.at[g]],
                rows_v.at[pl.ds(g * GATHER, GATHER)],
                sem,
            )

    def fire_gathers(idx_v, rows_v, sem):
        for c in gather_copies(idx_v, rows_v, sem):
            c.start()

    def wait_gathers(idx_v, rows_v, sem):
        for c in gather_copies(idx_v, rows_v, sem):
            c.wait()

    def xpose_block(rows_v):
        for sub in range(KH):
            t0 = sub * 128
            rv0 = rowp0 + sub * 8
            rv1 = rowp1 + sub * 8

            def body(il, carry):
                ilv = zero + il
                ta = t0 + il
                tb_ = ta + HB
                va0 = rows_v[ta, pl.ds(0, L)] * SCALE
                va1 = rows_v[ta, pl.ds(L, L)] * SCALE
                vb0 = rows_v[tb_, pl.ds(0, L)] * SCALE
                vb1 = rows_v[tb_, pl.ds(L, L)] * SCALE
                plsc.store_scatter(tb0, [rv0, ilv], va0)
                plsc.store_scatter(tb0, [rv1, ilv], va1)
                plsc.store_scatter(tb1, [rv0, ilv], vb0)
                plsc.store_scatter(tb1, [rv1, ilv], vb1)
                return carry

            lax.fori_loop(0, 128, body, 0, unroll=4)

    def out_copies(b, tb, h, sem):
        j = b // NIB
        ib = b % NIB
        for c4 in range(D // 8):
            yield pltpu.make_async_copy(
                tb.at[pl.ds(c4 * TROW, KH * 8), pl.ds(0, 128)],
                out_hbm.at[j, c4, pl.ds((ib * K + h * KH) * 8, KH * 8)],
                sem,
            )

    def fire_out(b, tb, h, sem):
        for c in out_copies(b, tb, h, sem):
            c.start()

    def wait_out(b, tb, h, sem):
        for c in out_copies(b, tb, h, sem):
            c.wait()

    def process(b, rows_v, first):
        @pl.when(jnp.logical_not(first))
        def _():
            wait_out(b, tb0, 0, sem_o0)
            wait_out(b, tb1, 1, sem_o1)

        xpose_block(rows_v)
        fire_out(b, tb0, 0, sem_o0)
        fire_out(b, tb1, 1, sem_o1)

    idx_copy(base, idx_a, sem_ia).start()
    idx_copy(base, idx_a, sem_ia).wait()
    fire_gathers(idx_a, rows_a, sem_a)
    idx_copy(base + 1, idx_b, sem_ib).start()

    def pair_body(k, carry):
        b0 = base + 2 * k
        b1 = b0 + 1
        not_last = k + 1 < NPAIR

        idx_copy(b1, idx_b, sem_ib).wait()
        fire_gathers(idx_b, rows_b, sem_b)

        wait_gathers(idx_a, rows_a, sem_a)

        @pl.when(not_last)
        def _():
            idx_copy(b0 + 2, idx_a, sem_ia).start()

        process(b0, rows_a, k == 0)

        @pl.when(not_last)
        def _():
            idx_copy(b0 + 2, idx_a, sem_ia).wait()
            fire_gathers(idx_a, rows_a, sem_a)

        wait_gathers(idx_b, rows_b, sem_b)

        @pl.when(not_last)
        def _():
            idx_copy(b1 + 2, idx_b, sem_ib).start()

        process(b1, rows_b, False)
        return carry

    lax.fori_loop(0, NPAIR, pair_body, 0)
    wait_out(0, tb0, 0, sem_o0)
    wait_out(0, tb1, 1, sem_o1)


def kernel(x, table):
    xt = x.T.reshape(NJ, NI // 128, 128)
    o4 = _embed(xt, table)
    return (
        o4.reshape(NJ, D // 8, NI // 128, 8, 128)
        .transpose(2, 4, 0, 1, 3)
        .reshape(NI, NJ, D)
    )


# speedup vs baseline: 1.1341x; 1.0115x over previous
import functools
import math

import jax
import jax.numpy as jnp
from jax import lax
from jax.experimental import pallas as pl
from jax.experimental.pallas import tpu as pltpu
from jax.experimental.pallas import tpu_sc as plsc

NI = 16384
NJ = 200
D = 32
N = NI * NJ
SCALE = math.sqrt(32.0)

_info = plsc.get_sparse_core_info()
NC = _info.num_cores
NS = _info.num_subcores
NW = NC * NS
L = _info.num_lanes

IB = 1024
HB = IB // 2
GATHER = 128
K = IB // GATHER
KH = K // 2
NBLK = NJ * (NI // IB)
PER_W = NBLK // NW
NPAIR = PER_W // 2
NIB = NI // IB

TROW = (KH + 1) * 8
TPITCH = 129

_mesh = plsc.VectorSubcoreMesh(core_axis_name="c", subcore_axis_name="s")


@functools.partial(
    pl.kernel,
    mesh=_mesh,
    out_type=jax.ShapeDtypeStruct((NJ, D // 8, (NI // 128) * 8, 128), jnp.float32),
    scratch_types=[
        pltpu.VMEM((K, GATHER), jnp.int32),
        pltpu.VMEM((K, GATHER), jnp.int32),
        pltpu.VMEM((IB, D), jnp.float32),
        pltpu.VMEM((IB, D), jnp.float32),
        pltpu.VMEM((4 * TROW, TPITCH), jnp.float32),
        pltpu.VMEM((4 * TROW, TPITCH), jnp.float32),
        pltpu.SemaphoreType.DMA,
        pltpu.SemaphoreType.DMA,
        pltpu.SemaphoreType.DMA,
        pltpu.SemaphoreType.DMA,
        pltpu.SemaphoreType.DMA,
        pltpu.SemaphoreType.DMA,
    ],
    compiler_params=pltpu.CompilerParams(
        use_tc_tiling_on_sc=False, needs_layout_passes=False
    ),
)
def _embed(
    xt_hbm, table_hbm, out_hbm,
    idx_a, idx_b, rows_a, rows_b, tb0, tb1,
    sem_a, sem_b, sem_o0, sem_o1, sem_ia, sem_ib,
):
    wid = lax.axis_index("s") * NC + lax.axis_index("c")
    base = wid * PER_W
    iota = lax.iota(jnp.int32, L)
    zero = iota * 0
    rowp0 = (iota // 8) * TROW + iota % 8
    rowp1 = rowp0 + 2 * TROW

    def idx_copy(b, idx_v, sem):
        j = b // NIB
        ib = b % NIB
        return pltpu.make_async_copy(
            xt_hbm.at[j, pl.ds(ib * K, K)], idx_v, sem
        )

    def gather_copies(idx_v, rows_v, sem):
        for g in range(K):
            yield pltpu.make_async_copy(
                table_hbm.at[idx_v.at[g]],
                rows_v.at[pl.ds(g * GATHER, GATHER)],
                sem,
            )

    def fire_gathers(idx_v, rows_v, sem):
        for c in gather_copies(idx_v, rows_v, sem):
            c.start()

    def wait_gathers(idx_v, rows_v, sem):
        for c in gather_copies(idx_v, rows_v, sem):
            c.wait()

    def xpose_half(rows_v, tb, h):
        for sub in range(KH):
            t0 = h * HB + sub * 128
            rv0 = rowp0 + sub * 8
            rv1 = rowp1 + sub * 8

            def body(il, carry):
                ilv = zero + il
                t = t0 + il
                val0 = rows_v[t, pl.ds(0, L)] * SCALE
                val1 = rows_v[t, pl.ds(L, L)] * SCALE
                plsc.store_scatter(tb, [rv0, ilv], val0)
                plsc.store_scatter(tb, [rv1, ilv], val1)
                return carry

            lax.fori_loop(0, 128, body, 0, unroll=4)

    def out_copies(b, tb, h, sem):
        j = b // NIB
        ib = b % NIB
        for c4 in range(D // 8):
            yield pltpu.make_async_copy(
                tb.at[pl.ds(c4 * TROW, KH * 8), pl.ds(0, 128)],
                out_hbm.at[j, c4, pl.ds((ib * K + h * KH) * 8, KH * 8)],
                sem,
            )

    def fire_out(b, tb, h, sem):
        for c in out_copies(b, tb, h, sem):
            c.start()

    def wait_out(b, tb, h, sem):
        for c in out_copies(b, tb, h, sem):
            c.wait()

    def process(b, rows_v, first):
        @pl.when(jnp.logical_not(first))
        def _():
            wait_out(b, tb0, 0, sem_o0)

        xpose_half(rows_v, tb0, 0)
        fire_out(b, tb0, 0, sem_o0)

        @pl.when(jnp.logical_not(first))
        def _():
            wait_out(b, tb1, 1, sem_o1)

        xpose_half(rows_v, tb1, 1)
        fire_out(b, tb1, 1, sem_o1)

    idx_copy(base, idx_a, sem_ia).start()
    idx_copy(base, idx_a, sem_ia).wait()
    fire_gathers(idx_a, rows_a, sem_a)
    idx_copy(base + 1, idx_b, sem_ib).start()

    def pair_body(k, carry):
        b0 = base + 2 * k
        b1 = b0 + 1
        not_last = k + 1 < NPAIR

        idx_copy(b1, idx_b, sem_ib).wait()
        fire_gathers(idx_b, rows_b, sem_b)

        wait_gathers(idx_a, rows_a, sem_a)

        @pl.when(not_last)
        def _():
            idx_copy(b0 + 2, idx_a, sem_ia).start()

        process(b0, rows_a, k == 0)

        @pl.when(not_last)
        def _():
            idx_copy(b0 + 2, idx_a, sem_ia).wait()
            fire_gathers(idx_a, rows_a, sem_a)

        wait_gathers(idx_b, rows_b, sem_b)

        @pl.when(not_last)
        def _():
            idx_copy(b1 + 2, idx_b, sem_ib).start()

        process(b1, rows_b, False)
        return carry

    lax.fori_loop(0, NPAIR, pair_body, 0)
    wait_out(0, tb0, 0, sem_o0)
    wait_out(0, tb1, 1, sem_o1)


def kernel(x, table):
    xt = x.T.reshape(NJ, NI // 128, 128)
    o4 = _embed(xt, table)
    return (
        o4.reshape(NJ, D // 8, NI // 128, 8, 128)
        .transpose(2, 4, 0, 1, 3)
        .reshape(NI, NJ, D)
    )


# speedup vs baseline: 1.6862x; 1.4869x over previous
import functools
import math

import jax
import jax.numpy as jnp
from jax import lax
from jax.experimental import pallas as pl
from jax.experimental.pallas import tpu as pltpu
from jax.experimental.pallas import tpu_sc as plsc

NI = 16384
NJ = 200
D = 32
N = NI * NJ
SCALE = math.sqrt(32.0)

_info = plsc.get_sparse_core_info()
NC = _info.num_cores
NS = _info.num_subcores
NW = NC * NS
L = _info.num_lanes

IB = 1024
HB = IB // 2
GATHER = 128
K = IB // GATHER
KH = K // 2
NBLK = NJ * (NI // IB)
PER_W = NBLK // NW
NPAIR = PER_W // 2
NIB = NI // IB

TROW = (KH + 1) * 8
TPITCH = 129

_mesh = plsc.VectorSubcoreMesh(core_axis_name="c", subcore_axis_name="s")


@functools.partial(
    pl.kernel,
    mesh=_mesh,
    out_type=jax.ShapeDtypeStruct((NJ, D // 8, (NI // 128) * 8, 128), jnp.float32),
    scratch_types=[
        pltpu.VMEM((K, GATHER), jnp.int32),
        pltpu.VMEM((K, GATHER), jnp.int32),
        pltpu.VMEM((IB, D), jnp.float32),
        pltpu.VMEM((IB, D), jnp.float32),
        pltpu.VMEM((4 * TROW, TPITCH), jnp.float32),
        pltpu.VMEM((4 * TROW, TPITCH), jnp.float32),
        pltpu.SemaphoreType.DMA,
        pltpu.SemaphoreType.DMA,
        pltpu.SemaphoreType.DMA,
        pltpu.SemaphoreType.DMA,
        pltpu.SemaphoreType.DMA,
        pltpu.SemaphoreType.DMA,
    ],
    compiler_params=pltpu.CompilerParams(
        use_tc_tiling_on_sc=False, needs_layout_passes=False
    ),
)
def _embed(
    xt_hbm, table_hbm, out_hbm,
    idx_a, idx_b, rows_a, rows_b, tb0, tb1,
    sem_a, sem_b, sem_o0, sem_o1, sem_ia, sem_ib,
):
    wid = lax.axis_index("s") * NC + lax.axis_index("c")
    base = wid * PER_W
    iota = lax.iota(jnp.int32, L)
    zero = iota * 0
    rowp0 = (iota // 8) * TROW + iota % 8
    rowp1 = rowp0 + 2 * TROW

    def idx_copy(b, idx_v, sem):
        j = b // NIB
        ib = b % NIB
        return pltpu.make_async_copy(
            xt_hbm.at[j, pl.ds(ib * K, K)], idx_v, sem
        )

    def gather_copies(idx_v, rows_v, sem):
        for g in range(K):
            yield pltpu.make_async_copy(
                table_hbm.at[idx_v.at[g]],
                rows_v.at[pl.ds(g * GATHER, GATHER)],
                sem,
            )

    def fire_gathers(idx_v, rows_v, sem):
        for c in gather_copies(idx_v, rows_v, sem):
            c.start()

    def wait_gathers(idx_v, rows_v, sem):
        for c in gather_copies(idx_v, rows_v, sem):
            c.wait()

    def xpose_half(rows_v, tb, h):
        for sub in range(KH):
            t0 = h * HB + sub * 128
            rv0 = rowp0 + sub * 8
            rv1 = rowp1 + sub * 8

            @plsc.parallel_loop(0, 128, 1, unroll=4)
            def _(il):
                ilv = zero + il
                t = t0 + il
                val0 = rows_v[t, pl.ds(0, L)] * SCALE
                val1 = rows_v[t, pl.ds(L, L)] * SCALE
                plsc.store_scatter(tb, [rv0, ilv], val0)
                plsc.store_scatter(tb, [rv1, ilv], val1)

    def out_copies(b, tb, h, sem):
        j = b // NIB
        ib = b % NIB
        for c4 in range(D // 8):
            yield pltpu.make_async_copy(
                tb.at[pl.ds(c4 * TROW, KH * 8), pl.ds(0, 128)],
                out_hbm.at[j, c4, pl.ds((ib * K + h * KH) * 8, KH * 8)],
                sem,
            )

    def fire_out(b, tb, h, sem):
        for c in out_copies(b, tb, h, sem):
            c.start()

    def wait_out(b, tb, h, sem):
        for c in out_copies(b, tb, h, sem):
            c.wait()

    def process(b, rows_v, first):
        @pl.when(jnp.logical_not(first))
        def _():
            wait_out(b, tb0, 0, sem_o0)

        xpose_half(rows_v, tb0, 0)
        fire_out(b, tb0, 0, sem_o0)

        @pl.when(jnp.logical_not(first))
        def _():
            wait_out(b, tb1, 1, sem_o1)

        xpose_half(rows_v, tb1, 1)
        fire_out(b, tb1, 1, sem_o1)

    idx_copy(base, idx_a, sem_ia).start()
    idx_copy(base, idx_a, sem_ia).wait()
    fire_gathers(idx_a, rows_a, sem_a)
    idx_copy(base + 1, idx_b, sem_ib).start()

    def pair_body(k, carry):
        b0 = base + 2 * k
        b1 = b0 + 1
        not_last = k + 1 < NPAIR

        idx_copy(b1, idx_b, sem_ib).wait()
        fire_gathers(idx_b, rows_b, sem_b)

        wait_gathers(idx_a, rows_a, sem_a)

        @pl.when(not_last)
        def _():
            idx_copy(b0 + 2, idx_a, sem_ia).start()

        process(b0, rows_a, k == 0)

        @pl.when(not_last)
        def _():
            idx_copy(b0 + 2, idx_a, sem_ia).wait()
            fire_gathers(idx_a, rows_a, sem_a)

        wait_gathers(idx_b, rows_b, sem_b)

        @pl.when(not_last)
        def _():
            idx_copy(b1 + 2, idx_b, sem_ib).start()

        process(b1, rows_b, False)
        return carry

    lax.fori_loop(0, NPAIR, pair_body, 0)
    wait_out(0, tb0, 0, sem_o0)
    wait_out(0, tb1, 1, sem_o1)


def kernel(x, table):
    xt = x.T.reshape(NJ, NI // 128, 128)
    o4 = _embed(xt, table)
    return (
        o4.reshape(NJ, D // 8, NI // 128, 8, 128)
        .transpose(2, 4, 0, 1, 3)
        .reshape(NI, NJ, D)
    )
